# Initial kernel scaffold; baseline (speedup 1.0000x reference)
import jax
import jax.numpy as jnp
from jax.experimental import pallas as pl


def kernel(x, edge_index, W):
    raise NotImplementedError("write your pallas kernel here")


# speedup vs baseline: 15.6786x; 15.6786x over previous
import functools

import jax
import jax.numpy as jnp
from jax import lax
from jax.experimental import pallas as pl
from jax.experimental.pallas import tpu as pltpu
from jax.experimental.pallas import tpu_sc as plsc

N = 10000
E = 320000
D = 128
C = 64

NC = 2
NS = 16
NW = NC * NS
EPW = E // NW
CHUNK = 128
NFULL = EPW // CHUNK
REM = EPW - NFULL * CHUNK
NPAD = 10240
SLICE = NPAD // NS

_MESH = plsc.VectorSubcoreMesh(
    core_axis_name="c", subcore_axis_name="s", num_cores=NC, num_subcores=NS
)


@functools.partial(
    pl.kernel,
    out_type=jax.ShapeDtypeStruct((NC, NPAD), jnp.float32),
    mesh=_MESH,
    scratch_types=[
        pltpu.VMEM((CHUNK,), jnp.int32),
        pltpu.VMEM((REM,), jnp.int32),
        pltpu.VMEM((CHUNK,), jnp.float32),
        pltpu.VMEM((SLICE,), jnp.float32),
        pltpu.VMEM_SHARED((NPAD,), jnp.float32),
    ],
)
def _deg_kernel(col_hbm, out_hbm, idx_v, idx16_v, ones_v, zeros_v, acc_sh):
    cid = lax.axis_index("c")
    sid = lax.axis_index("s")
    wid = cid * NS + sid
    for i in range(CHUNK // 16):
        ones_v[pl.ds(i * 16, 16)] = jnp.ones((16,), jnp.float32)
    for i in range(SLICE // 16):
        zeros_v[pl.ds(i * 16, 16)] = jnp.zeros((16,), jnp.float32)
    pltpu.sync_copy(zeros_v, acc_sh.at[pl.ds(sid * SLICE, SLICE)])
    plsc.subcore_barrier()

    ebase = wid * EPW

    def body(j, carry):
        pltpu.sync_copy(col_hbm.at[pl.ds(ebase + j * CHUNK, CHUNK)], idx_v)
        pltpu.sync_copy(ones_v, acc_sh.at[idx_v], add=True)
        return carry

    lax.fori_loop(0, NFULL, body, 0)
    pltpu.sync_copy(col_hbm.at[pl.ds(ebase + NFULL * CHUNK, REM)], idx16_v)
    pltpu.sync_copy(ones_v.at[pl.ds(0, REM)], acc_sh.at[idx16_v], add=True)

    plsc.subcore_barrier()
    pltpu.sync_copy(
        acc_sh.at[pl.ds(sid * SLICE, SLICE)],
        out_hbm.at[cid, pl.ds(sid * SLICE, SLICE)],
    )


@functools.partial(
    pl.kernel,
    out_type=jax.ShapeDtypeStruct((NC, NPAD, C), jnp.float32),
    mesh=_MESH,
    scratch_types=[
        pltpu.VMEM((CHUNK,), jnp.int32),
        pltpu.VMEM((CHUNK,), jnp.int32),
        pltpu.VMEM((REM,), jnp.int32),
        pltpu.VMEM((REM,), jnp.int32),
        pltpu.VMEM((CHUNK, C), jnp.float32),
        pltpu.VMEM((REM, C), jnp.float32),
        pltpu.VMEM_SHARED((NPAD, C), jnp.float32),
    ],
    compiler_params=pltpu.CompilerParams(use_tc_tiling_on_sc=False),
)
def _prop_kernel(
    row_hbm, col_hbm, z_hbm, zblk_hbm, out_hbm,
    ridx_v, cidx_v, ridx16_v, cidx16_v, rows_v, rows16_v, acc_sh,
):
    cid = lax.axis_index("c")
    sid = lax.axis_index("s")
    wid = cid * NS + sid
    pltpu.sync_copy(zblk_hbm, acc_sh.at[pl.ds(sid * SLICE, SLICE)])
    plsc.subcore_barrier()

    ebase = wid * EPW

    def body(j, carry):
        off = ebase + j * CHUNK
        pltpu.sync_copy(row_hbm.at[pl.ds(off, CHUNK)], ridx_v)
        pltpu.sync_copy(col_hbm.at[pl.ds(off, CHUNK)], cidx_v)
        pltpu.sync_copy(z_hbm.at[ridx_v], rows_v)
        pltpu.sync_copy(rows_v, acc_sh.at[cidx_v], add=True)
        return carry

    lax.fori_loop(0, NFULL, body, 0)
    off = ebase + NFULL * CHUNK
    pltpu.sync_copy(row_hbm.at[pl.ds(off, REM)], ridx16_v)
    pltpu.sync_copy(col_hbm.at[pl.ds(off, REM)], cidx16_v)
    pltpu.sync_copy(z_hbm.at[ridx16_v], rows16_v)
    pltpu.sync_copy(rows16_v, acc_sh.at[cidx16_v], add=True)

    plsc.subcore_barrier()
    pltpu.sync_copy(
        acc_sh.at[pl.ds(sid * SLICE, SLICE)],
        out_hbm.at[cid, pl.ds(sid * SLICE, SLICE)],
    )


def _mm_body(x_ref, wt_ref, s_ref, o_ref):
    o_ref[...] = (
        jnp.dot(x_ref[...], wt_ref[...], preferred_element_type=jnp.float32)
        * s_ref[...]
    )


_MM_BLK = 1000
_mm_call = pl.pallas_call(
    _mm_body,
    grid=(N // _MM_BLK,),
    in_specs=[
        pl.BlockSpec((_MM_BLK, D), lambda i: (i, 0)),
        pl.BlockSpec((D, C), lambda i: (0, 0)),
        pl.BlockSpec((_MM_BLK, 1), lambda i: (i, 0)),
    ],
    out_specs=pl.BlockSpec((_MM_BLK, C), lambda i: (i, 0)),
    out_shape=jax.ShapeDtypeStruct((N, C), jnp.float32),
)


def _merge_body(p_ref, s_ref, o_ref):
    o_ref[...] = (p_ref[0] + p_ref[1]) * s_ref[...]


_MG_BLK = 1024
_merge_call = pl.pallas_call(
    _merge_body,
    grid=(NPAD // _MG_BLK,),
    in_specs=[
        pl.BlockSpec((2, _MG_BLK, C), lambda i: (0, i, 0)),
        pl.BlockSpec((_MG_BLK, 1), lambda i: (i, 0)),
    ],
    out_specs=pl.BlockSpec((_MG_BLK, C), lambda i: (i, 0)),
    out_shape=jax.ShapeDtypeStruct((NPAD, C), jnp.float32),
)


def kernel(x, edge_index, W):
    row = edge_index[0]
    col = edge_index[1]

    degp = _deg_kernel(col)
    deg = degp[0, :N] + degp[1, :N]
    dinv = jnp.where(deg > 0.0, lax.rsqrt(deg), 0.0)
    dinv_pad = jnp.pad(dinv, (0, NPAD - N))

    z0 = _mm_call(x, W.T, dinv[:, None])
    zblk = jnp.zeros((SLICE, C), jnp.float32)

    p1 = _prop_kernel(row, col, z0, zblk)
    z1 = _merge_call(p1, (dinv_pad * dinv_pad)[:, None])
    p2 = _prop_kernel(row, col, z1, zblk)
    out_pad = _merge_call(p2, dinv_pad[:, None])
    return out_pad[:N]
